# baseline (speedup 1.0000x reference)
import functools

import jax
import jax.numpy as jnp
from jax.experimental import pallas as pl
from jax.experimental.pallas import tpu as pltpu
from jax.experimental.pallas import tpu_sc as plsc

DIM = 64
HIDDEN = 128

VOCAB_BLOCK = 16384
GATHER_WINDOW = 128


def _proj_body(tt_ref, wt_ref, out_ref):
    t = jnp.transpose(tt_ref[...])
    wt = wt_ref[...]

    def _split(a):
        hi = a.astype(jnp.bfloat16)
        lo = (a - hi.astype(jnp.float32)).astype(jnp.bfloat16)
        return hi, lo

    t_hi, t_lo = _split(t)
    w_hi, w_lo = _split(wt)
    dims = (((1,), (0,)), ((), ()))

    def _mm(a, b):
        return jax.lax.dot_general(a, b, dims,
                                   preferred_element_type=jnp.float32)

    h = _mm(t_hi, w_hi) + (_mm(t_lo, w_hi) + _mm(t_hi, w_lo))

    hb = h.astype(jnp.bfloat16)
    ss = jax.lax.dot_general(
        hb * hb,
        jnp.ones((HIDDEN, 8), jnp.bfloat16),
        dims,
        preferred_element_type=jnp.float32,
    )
    rinv = jax.lax.rsqrt(jnp.maximum(ss[:, :1], 1e-24))
    out_ref[...] = h * jax.lax.broadcast_in_dim(
        rinv, (h.shape[0], HIDDEN), (0, 1)
    )


def _tc_project_all(table_t, w_t):
    vocab = table_t.shape[1]
    return pl.pallas_call(
        _proj_body,
        grid=(pl.cdiv(vocab, VOCAB_BLOCK),),
        in_specs=[
            pl.BlockSpec((DIM, VOCAB_BLOCK), lambda i: (0, i)),
            pl.BlockSpec((DIM, HIDDEN), lambda i: (0, 0)),
        ],
        out_specs=pl.BlockSpec((VOCAB_BLOCK, HIDDEN), lambda i: (i, 0)),
        out_shape=jax.ShapeDtypeStruct((vocab, HIDDEN), jnp.float32),
        compiler_params=pltpu.CompilerParams(
            dimension_semantics=("parallel",)
        ),
    )(table_t, w_t)


def _sc_gather(rows, idx_flat):
    n = idx_flat.shape[1]
    mesh = plsc.VectorSubcoreMesh(core_axis_name="c", subcore_axis_name="s")

    @functools.partial(
        pl.kernel,
        out_type=jax.ShapeDtypeStruct((n, HIDDEN), rows.dtype),
        mesh=mesh,
    )
    def gather_kernel(rows_hbm, idx_hbm, out_hbm):
        def body(idx_vmem, out_vmem):
            pltpu.sync_copy(rows_hbm.at[idx_vmem.at[0]], out_vmem)

        pltpu.emit_pipeline(
            body,
            grid=(n // GATHER_WINDOW,),
            in_specs=[
                pl.BlockSpec((1, GATHER_WINDOW), index_map=lambda i: (0, i))
            ],
            out_specs=[
                pl.BlockSpec((GATHER_WINDOW, HIDDEN), index_map=lambda i: (i, 0))
            ],
            core_axis_name=("c", "s"),
            dimension_semantics=(pltpu.PARALLEL,),
        )(idx_hbm, out_hbm)

    return gather_kernel(rows, idx_flat)


def kernel(x, table, W):
    batch, fields = x.shape
    n = batch * fields
    hidden_norm = _tc_project_all(table.T, W.T)
    idx_flat = x.T.reshape(1, n)
    out = _sc_gather(hidden_norm, idx_flat)
    return out.reshape(fields, batch, HIDDEN).transpose(1, 0, 2)


# speedup vs baseline: 1.0262x; 1.0262x over previous
import functools

import jax
import jax.numpy as jnp
from jax.experimental import pallas as pl
from jax.experimental.pallas import tpu as pltpu
from jax.experimental.pallas import tpu_sc as plsc

DIM = 64
HIDDEN = 128

VOCAB_BLOCK = 24576
GATHER_WINDOW = 128


def _proj_body(tt_ref, wt_ref, out_ref):
    t = jnp.transpose(tt_ref[...])
    wt = wt_ref[...]

    def _split(a):
        hi = a.astype(jnp.bfloat16)
        lo = (a - hi.astype(jnp.float32)).astype(jnp.bfloat16)
        return hi, lo

    t_hi, t_lo = _split(t)
    w_hi, w_lo = _split(wt)
    dims = (((1,), (0,)), ((), ()))

    def _mm(a, b):
        return jax.lax.dot_general(a, b, dims,
                                   preferred_element_type=jnp.float32)

    h = _mm(t_hi, w_hi) + (_mm(t_lo, w_hi) + _mm(t_hi, w_lo))

    hb = h.astype(jnp.bfloat16)
    ss = jax.lax.dot_general(
        hb * hb,
        jnp.ones((HIDDEN, 8), jnp.bfloat16),
        dims,
        preferred_element_type=jnp.float32,
    )
    rinv = jax.lax.rsqrt(jnp.maximum(ss[:, :1], 1e-24))
    out_ref[...] = (h * jax.lax.broadcast_in_dim(
        rinv, (h.shape[0], HIDDEN), (0, 1)
    ))


def _tc_project_all(table_t, w_t):
    vocab = table_t.shape[1]
    return pl.pallas_call(
        _proj_body,
        grid=(pl.cdiv(vocab, VOCAB_BLOCK),),
        in_specs=[
            pl.BlockSpec((DIM, VOCAB_BLOCK), lambda i: (0, i)),
            pl.BlockSpec((DIM, HIDDEN), lambda i: (0, 0)),
        ],
        out_specs=pl.BlockSpec((VOCAB_BLOCK, HIDDEN), lambda i: (i, 0)),
        out_shape=jax.ShapeDtypeStruct((vocab, HIDDEN), jnp.float32),
        compiler_params=pltpu.CompilerParams(
            dimension_semantics=("parallel",)
        ),
    )(table_t, w_t)


def _sc_gather(rows, idx_flat):
    n = idx_flat.shape[1]
    mesh = plsc.VectorSubcoreMesh(core_axis_name="c", subcore_axis_name="s")

    @functools.partial(
        pl.kernel,
        out_type=jax.ShapeDtypeStruct((n, HIDDEN), rows.dtype),
        mesh=mesh,
    )
    def gather_kernel(rows_hbm, idx_hbm, out_hbm):
        def body(idx_vmem, out_vmem):
            pltpu.sync_copy(rows_hbm.at[idx_vmem.at[0]], out_vmem)

        pltpu.emit_pipeline(
            body,
            grid=(n // GATHER_WINDOW,),
            in_specs=[
                pl.BlockSpec((1, GATHER_WINDOW), index_map=lambda i: (0, i))
            ],
            out_specs=[
                pl.BlockSpec((GATHER_WINDOW, HIDDEN), index_map=lambda i: (i, 0))
            ],
            core_axis_name=("c", "s"),
            dimension_semantics=(pltpu.PARALLEL,),
        )(idx_hbm, out_hbm)

    return gather_kernel(rows, idx_flat)


def kernel(x, table, W):
    batch, fields = x.shape
    n = batch * fields
    hidden_norm = _tc_project_all(table.T, W.T)
    idx_flat = x.T.reshape(1, n)
    out = _sc_gather(hidden_norm, idx_flat)
    return out.reshape(fields, batch, HIDDEN).transpose(1, 0, 2)


# speedup vs baseline: 1.0579x; 1.0309x over previous
import functools

import jax
import jax.numpy as jnp
from jax.experimental import pallas as pl
from jax.experimental.pallas import tpu as pltpu
from jax.experimental.pallas import tpu_sc as plsc

DIM = 64
HIDDEN = 128

VOCAB_BLOCK = 24576
GATHER_WINDOW = 128


def _proj_body(tt_ref, wt_ref, out_ref):
    t = jnp.transpose(tt_ref[...])
    wt = wt_ref[...]

    def _split(a):
        hi = a.astype(jnp.bfloat16)
        lo = (a - hi.astype(jnp.float32)).astype(jnp.bfloat16)
        return hi, lo

    t_hi, t_lo = _split(t)
    w_hi, w_lo = _split(wt)
    dims = (((1,), (0,)), ((), ()))

    def _mm(a, b):
        return jax.lax.dot_general(a, b, dims,
                                   preferred_element_type=jnp.float32)

    h = _mm(t_hi, w_hi) + (_mm(t_lo, w_hi) + _mm(t_hi, w_lo))

    hb = h.astype(jnp.bfloat16)
    ss = jax.lax.dot_general(
        hb * hb,
        jnp.ones((HIDDEN, 8), jnp.bfloat16),
        dims,
        preferred_element_type=jnp.float32,
    )
    rinv = jax.lax.rsqrt(jnp.maximum(ss[:, :1], 1e-24))
    out_ref[...] = (h * jax.lax.broadcast_in_dim(
        rinv, (h.shape[0], HIDDEN), (0, 1)
    ))


def _tc_project_all(table_t, w_t):
    vocab = table_t.shape[1]
    return pl.pallas_call(
        _proj_body,
        grid=(pl.cdiv(vocab, VOCAB_BLOCK),),
        in_specs=[
            pl.BlockSpec((DIM, VOCAB_BLOCK), lambda i: (0, i)),
            pl.BlockSpec((DIM, HIDDEN), lambda i: (0, 0)),
        ],
        out_specs=pl.BlockSpec((VOCAB_BLOCK, HIDDEN), lambda i: (i, 0)),
        out_shape=jax.ShapeDtypeStruct((vocab, HIDDEN), jnp.float32),
        compiler_params=pltpu.CompilerParams(
            dimension_semantics=("parallel",)
        ),
    )(table_t, w_t)


GATHER_BATCH = 2


def _sc_gather(rows, idx_2d):
    n = idx_2d.shape[0] * idx_2d.shape[1]
    step_rows = GATHER_BATCH * GATHER_WINDOW
    mesh = plsc.VectorSubcoreMesh(core_axis_name="c", subcore_axis_name="s")

    @functools.partial(
        pl.kernel,
        out_type=jax.ShapeDtypeStruct((n, HIDDEN), rows.dtype),
        mesh=mesh,
    )
    def gather_kernel(rows_hbm, idx_hbm, out_hbm):
        def body(idx_vmem, out_vmem):
            for j in range(GATHER_BATCH):
                pltpu.sync_copy(
                    rows_hbm.at[idx_vmem.at[j]],
                    out_vmem.at[pl.ds(j * GATHER_WINDOW, GATHER_WINDOW)],
                )

        pltpu.emit_pipeline(
            body,
            grid=(n // step_rows,),
            in_specs=[
                pl.BlockSpec(
                    (GATHER_BATCH, GATHER_WINDOW), index_map=lambda i: (i, 0)
                )
            ],
            out_specs=[
                pl.BlockSpec((step_rows, HIDDEN), index_map=lambda i: (i, 0))
            ],
            core_axis_name=("c", "s"),
            dimension_semantics=(pltpu.PARALLEL,),
        )(idx_hbm, out_hbm)

    return gather_kernel(rows, idx_2d)


def kernel(x, table, W):
    batch, fields = x.shape
    n = batch * fields
    hidden_norm = _tc_project_all(table.T, W.T)
    idx_2d = x.T.reshape(n // GATHER_WINDOW, GATHER_WINDOW)
    out = _sc_gather(hidden_norm, idx_2d)
    return out.reshape(fields, batch, HIDDEN).transpose(1, 0, 2)
